# baseline (speedup 1.0000x reference)
import functools

import jax
import jax.numpy as jnp
from jax import lax
from jax.experimental import pallas as pl
from jax.experimental.pallas import tpu as pltpu
from jax.experimental.pallas import tpu_sc as plsc

_N = 10000
_E = 320000
_D = 128

_NC = 2
_NS = 16
_NW = _NC * _NS
_CH = 96
_NCHUNK = 106
_EPW = _NCHUNK * _CH
_EP = _NW * _EPW
_SLAB = 624
_ZCH = 128
_TAIL = _N - _NS * _SLAB


def _mm_body(x_ref, w_ref, o_ref):
    o_ref[...] = jnp.dot(x_ref[...], w_ref[...],
                         preferred_element_type=jnp.float32)


def _matmul(x, W):
    return pl.pallas_call(
        _mm_body,
        grid=(10,),
        in_specs=[
            pl.BlockSpec((1000, _D), lambda i: (i, 0)),
            pl.BlockSpec((_D, _D), lambda i: (0, 0)),
        ],
        out_specs=pl.BlockSpec((1000, _D), lambda i: (i, 0)),
        out_shape=jax.ShapeDtypeStruct((_N, _D), jnp.float32),
    )(x, W)


def _add_body(a_ref, b_ref, o_ref):
    o_ref[...] = a_ref[...] + b_ref[...]


def _combine(p0, p1):
    return pl.pallas_call(
        _add_body,
        grid=(10,),
        in_specs=[
            pl.BlockSpec((1000, _D), lambda i: (i, 0)),
            pl.BlockSpec((1000, _D), lambda i: (i, 0)),
        ],
        out_specs=pl.BlockSpec((1000, _D), lambda i: (i, 0)),
        out_shape=jax.ShapeDtypeStruct((_N, _D), jnp.float32),
    )(p0, p1)


_mesh = plsc.VectorSubcoreMesh(core_axis_name="c", subcore_axis_name="s")


@functools.partial(
    pl.kernel,
    mesh=_mesh,
    out_type=jax.ShapeDtypeStruct((_NC, _N, _D), jnp.float32),
    scratch_types=[
        pltpu.VMEM((_CH,), jnp.int32),
        pltpu.VMEM((_CH,), jnp.int32),
        pltpu.VMEM((_CH,), jnp.int32),
        pltpu.VMEM((_CH,), jnp.int32),
        pltpu.VMEM((_CH,), jnp.float32),
        pltpu.VMEM((_CH,), jnp.float32),
        pltpu.VMEM((_CH, _D), jnp.float32),
        pltpu.VMEM((_CH, _D), jnp.float32),
        pltpu.VMEM_SHARED((_N, _D), jnp.float32),
        pltpu.SemaphoreType.DMA,
        pltpu.SemaphoreType.DMA,
        pltpu.SemaphoreType.DMA,
        pltpu.SemaphoreType.DMA,
    ],
)
def _sc_spmm(sup_hbm, src_hbm, dst_hbm, ev_hbm, out_hbm,
             srcv0, srcv1, dstv0, dstv1, evv0, evv1, rows0, rows1, acc,
             gsem0, gsem1, isem0, isem1):
    c = lax.axis_index("c")
    s = lax.axis_index("s")
    wid = c * _NS + s
    srcv = (srcv0, srcv1)
    dstv = (dstv0, dstv1)
    evv = (evv0, evv1)
    rows = (rows0, rows1)
    gsem = (gsem0, gsem1)
    isem = (isem0, isem1)

    def idx_start(k, b):
        base = wid * _EPW + k * _CH
        pltpu.async_copy(src_hbm.at[pl.ds(base, _CH)], srcv[b], isem[b])
        pltpu.async_copy(dst_hbm.at[pl.ds(base, _CH)], dstv[b], isem[b])
        pltpu.async_copy(ev_hbm.at[pl.ds(base, _CH)], evv[b], isem[b])

    def idx_wait(k, b):
        base = wid * _EPW + k * _CH
        pltpu.make_async_copy(src_hbm.at[pl.ds(base, _CH)], srcv[b],
                              isem[b]).wait()
        pltpu.make_async_copy(dst_hbm.at[pl.ds(base, _CH)], dstv[b],
                              isem[b]).wait()
        pltpu.make_async_copy(ev_hbm.at[pl.ds(base, _CH)], evv[b],
                              isem[b]).wait()

    def zb_body(i, carry):
        for b in range(_D // 16):
            rows0[i, pl.ds(b * 16, 16)] = jnp.zeros((16,), jnp.float32)
        return carry

    lax.fori_loop(0, _CH, zb_body, 0)
    for kz in range(_SLAB // _CH):
        pltpu.sync_copy(rows0, acc.at[pl.ds(s * _SLAB + kz * _CH, _CH)])
    pltpu.sync_copy(rows0.at[pl.ds(0, _SLAB - (_SLAB // _CH) * _CH)],
                    acc.at[pl.ds(s * _SLAB + (_SLAB // _CH) * _CH,
                                 _SLAB - (_SLAB // _CH) * _CH)])

    @pl.when(s == _NS - 1)
    def _zero_tail():
        pltpu.sync_copy(rows0.at[pl.ds(0, _TAIL)],
                        acc.at[pl.ds(_NS * _SLAB, _TAIL)])

    plsc.subcore_barrier()

    idx_start(0, 0)
    idx_wait(0, 0)
    pltpu.async_copy(sup_hbm.at[srcv0], rows[0], gsem0)
    idx_start(1, 1)

    def pair_body(kk, carry):
        for b in range(2):
            k = kk * 2 + b
            o = 1 - b
            pltpu.make_async_copy(sup_hbm.at[srcv[b]], rows[b],
                                  gsem[b]).wait()

            @pl.when(k + 1 < _NCHUNK)
            def _next_gather():
                idx_wait(k + 1, o)
                pltpu.async_copy(sup_hbm.at[srcv[o]], rows[o], gsem[o])

            def mul_body(g, inner):
                evg = evv[b][pl.ds(g * 16, 16)]
                for j in range(16):
                    v = evg[j]
                    e = g * 16 + j
                    for blk in range(_D // 16):
                        sl = rows[b][e, pl.ds(blk * 16, 16)]
                        rows[b][e, pl.ds(blk * 16, 16)] = sl * v
                return inner

            lax.fori_loop(0, _CH // 16, mul_body, 0)

            pltpu.sync_copy(rows[b], acc.at[dstv[b]], add=True)

            @pl.when(k + 2 < _NCHUNK)
            def _next_idx():
                idx_start(k + 2, b)

        return carry

    lax.fori_loop(0, _NCHUNK // 2, pair_body, 0)
    plsc.subcore_barrier()

    pltpu.sync_copy(acc.at[pl.ds(s * _SLAB, _SLAB)],
                    out_hbm.at[c, pl.ds(s * _SLAB, _SLAB)])

    @pl.when(s == _NS - 1)
    def _copy_tail():
        pltpu.sync_copy(acc.at[pl.ds(_NS * _SLAB, _TAIL)],
                        out_hbm.at[c, pl.ds(_NS * _SLAB, _TAIL)])


def kernel(x, edge_index, edge_vals, W):
    support = _matmul(x, W)
    dst = edge_index[0]
    src = edge_index[1]
    pad = _EP - _E
    srcp = jnp.concatenate([src, jnp.zeros((pad,), jnp.int32)])
    dstp = jnp.concatenate([dst, jnp.zeros((pad,), jnp.int32)])
    evp = jnp.concatenate([edge_vals, jnp.zeros((pad,), jnp.float32)])
    partials = _sc_spmm(support, srcp, dstp, evp)
    return _combine(partials[0], partials[1])


# speedup vs baseline: 1.5255x; 1.5255x over previous
import functools

import jax
import jax.numpy as jnp
from jax import lax
from jax.experimental import pallas as pl
from jax.experimental.pallas import tpu as pltpu
from jax.experimental.pallas import tpu_sc as plsc

_N = 10000
_E = 320000
_D = 128

_NC = 2
_NS = 16
_NW = _NC * _NS
_CH = 80
_NCHUNK = 126
_EPW = _NCHUNK * _CH
_EP = _NW * _EPW
_SLAB = 624
_ZCH = 128
_TAIL = _N - _NS * _SLAB


def _mm_body(x_ref, w_ref, o_ref):
    o_ref[...] = jnp.dot(x_ref[...], w_ref[...],
                         preferred_element_type=jnp.float32)


def _matmul(x, W):
    return pl.pallas_call(
        _mm_body,
        grid=(10,),
        in_specs=[
            pl.BlockSpec((1000, _D), lambda i: (i, 0)),
            pl.BlockSpec((_D, _D), lambda i: (0, 0)),
        ],
        out_specs=pl.BlockSpec((1000, _D), lambda i: (i, 0)),
        out_shape=jax.ShapeDtypeStruct((_N, _D), jnp.float32),
    )(x, W)


def _add_body(a_ref, b_ref, o_ref):
    o_ref[...] = a_ref[...] + b_ref[...]


def _combine(p0, p1):
    return pl.pallas_call(
        _add_body,
        grid=(10,),
        in_specs=[
            pl.BlockSpec((1000, _D), lambda i: (i, 0)),
            pl.BlockSpec((1000, _D), lambda i: (i, 0)),
        ],
        out_specs=pl.BlockSpec((1000, _D), lambda i: (i, 0)),
        out_shape=jax.ShapeDtypeStruct((_N, _D), jnp.float32),
    )(p0, p1)


_mesh = plsc.VectorSubcoreMesh(core_axis_name="c", subcore_axis_name="s")


@functools.partial(
    pl.kernel,
    mesh=_mesh,
    out_type=jax.ShapeDtypeStruct((_NC, _N, _D), jnp.float32),
    scratch_types=[
        pltpu.VMEM((_CH,), jnp.int32),
        pltpu.VMEM((_CH,), jnp.int32),
        pltpu.VMEM((_CH,), jnp.int32),
        pltpu.VMEM((_CH,), jnp.int32),
        pltpu.VMEM((_CH,), jnp.int32),
        pltpu.VMEM((_CH,), jnp.int32),
        pltpu.VMEM((_CH,), jnp.float32),
        pltpu.VMEM((_CH,), jnp.float32),
        pltpu.VMEM((_CH,), jnp.float32),
        pltpu.VMEM((_CH, _D), jnp.float32),
        pltpu.VMEM((_CH, _D), jnp.float32),
        pltpu.VMEM((_CH, _D), jnp.float32),
        pltpu.VMEM_SHARED((_N, _D), jnp.float32),
        pltpu.SemaphoreType.DMA,
        pltpu.SemaphoreType.DMA,
        pltpu.SemaphoreType.DMA,
        pltpu.SemaphoreType.DMA,
        pltpu.SemaphoreType.DMA,
        pltpu.SemaphoreType.DMA,
    ],
)
def _sc_spmm(sup_hbm, src_hbm, dst_hbm, ev_hbm, out_hbm,
             srcv0, srcv1, srcv2, dstv0, dstv1, dstv2, evv0, evv1, evv2,
             rows0, rows1, rows2, acc,
             gsem0, gsem1, gsem2, isem0, isem1, isem2):
    c = lax.axis_index("c")
    s = lax.axis_index("s")
    wid = c * _NS + s
    srcv = (srcv0, srcv1, srcv2)
    dstv = (dstv0, dstv1, dstv2)
    evv = (evv0, evv1, evv2)
    rows = (rows0, rows1, rows2)
    gsem = (gsem0, gsem1, gsem2)
    isem = (isem0, isem1, isem2)

    def idx_start(k, b):
        base = wid * _EPW + k * _CH
        pltpu.async_copy(src_hbm.at[pl.ds(base, _CH)], srcv[b], isem[b])
        pltpu.async_copy(dst_hbm.at[pl.ds(base, _CH)], dstv[b], isem[b])
        pltpu.async_copy(ev_hbm.at[pl.ds(base, _CH)], evv[b], isem[b])

    def idx_wait(k, b):
        base = wid * _EPW + k * _CH
        pltpu.make_async_copy(src_hbm.at[pl.ds(base, _CH)], srcv[b],
                              isem[b]).wait()
        pltpu.make_async_copy(dst_hbm.at[pl.ds(base, _CH)], dstv[b],
                              isem[b]).wait()
        pltpu.make_async_copy(ev_hbm.at[pl.ds(base, _CH)], evv[b],
                              isem[b]).wait()

    def zb_body(i, carry):
        for b in range(_D // 16):
            rows0[i, pl.ds(b * 16, 16)] = jnp.zeros((16,), jnp.float32)
        return carry

    lax.fori_loop(0, _CH, zb_body, 0)
    for kz in range(_SLAB // _CH):
        pltpu.sync_copy(rows0, acc.at[pl.ds(s * _SLAB + kz * _CH, _CH)])
    pltpu.sync_copy(rows0.at[pl.ds(0, _SLAB - (_SLAB // _CH) * _CH)],
                    acc.at[pl.ds(s * _SLAB + (_SLAB // _CH) * _CH,
                                 _SLAB - (_SLAB // _CH) * _CH)])

    @pl.when(s == _NS - 1)
    def _zero_tail():
        pltpu.sync_copy(rows0.at[pl.ds(0, _TAIL)],
                        acc.at[pl.ds(_NS * _SLAB, _TAIL)])

    plsc.subcore_barrier()

    idx_start(0, 0)
    idx_wait(0, 0)
    pltpu.async_copy(sup_hbm.at[srcv0], rows0, gsem0)
    idx_start(1, 1)
    idx_wait(1, 1)
    pltpu.async_copy(sup_hbm.at[srcv1], rows1, gsem1)
    idx_start(2, 2)

    def trip_body(kk, carry):
        for b in range(3):
            k = kk * 3 + b
            b2 = (b + 2) % 3
            pltpu.make_async_copy(sup_hbm.at[srcv[b]], rows[b],
                                  gsem[b]).wait()

            @pl.when(k + 2 < _NCHUNK)
            def _next_gather():
                idx_wait(k + 2, b2)
                pltpu.async_copy(sup_hbm.at[srcv[b2]], rows[b2], gsem[b2])

            def mul_body(g, inner):
                evg = evv[b][pl.ds(g * 16, 16)]
                for j in range(16):
                    v = evg[j]
                    e = g * 16 + j
                    for blk in range(_D // 16):
                        sl = rows[b][e, pl.ds(blk * 16, 16)]
                        rows[b][e, pl.ds(blk * 16, 16)] = sl * v
                return inner

            lax.fori_loop(0, _CH // 16, mul_body, 0)

            pltpu.sync_copy(rows[b], acc.at[dstv[b]], add=True)

            @pl.when(k + 3 < _NCHUNK)
            def _next_idx():
                idx_start(k + 3, b)

        return carry

    lax.fori_loop(0, _NCHUNK // 3, trip_body, 0)
    plsc.subcore_barrier()

    pltpu.sync_copy(acc.at[pl.ds(s * _SLAB, _SLAB)],
                    out_hbm.at[c, pl.ds(s * _SLAB, _SLAB)])

    @pl.when(s == _NS - 1)
    def _copy_tail():
        pltpu.sync_copy(acc.at[pl.ds(_NS * _SLAB, _TAIL)],
                        out_hbm.at[c, pl.ds(_NS * _SLAB, _TAIL)])


def kernel(x, edge_index, edge_vals, W):
    support = _matmul(x, W)
    dst = edge_index[0]
    src = edge_index[1]
    pad = _EP - _E
    srcp = jnp.concatenate([src, jnp.zeros((pad,), jnp.int32)])
    dstp = jnp.concatenate([dst, jnp.zeros((pad,), jnp.int32)])
    evp = jnp.concatenate([edge_vals, jnp.zeros((pad,), jnp.float32)])
    partials = _sc_spmm(support, srcp, dstp, evp)
    return _combine(partials[0], partials[1])
